# Initial kernel scaffold; baseline (speedup 1.0000x reference)
import jax
import jax.numpy as jnp
from jax.experimental import pallas as pl


def kernel(x, edge_index, batch, W1, b1, W2, b2, W3, b3, Wl1, bl1, Wl2, bl2):
    raise NotImplementedError("write your pallas kernel here")


# speedup vs baseline: 18.1242x; 18.1242x over previous
import functools

import jax
import jax.numpy as jnp
from jax import lax
from jax.experimental import pallas as pl
from jax.experimental.pallas import tpu as pltpu
from jax.experimental.pallas import tpu_sc as plsc

_N = 10000
_E = 320000
_D = 128
_G = 128
_C = 16

_NC = 2
_NS = 16
_NW = _NC * _NS
_CHUNK = 128
_CH = 80
_PW = _CH * _CHUNK
_EPAD = _NW * _PW
_NROWS = 10240
_STRIPE = _NROWS // _NS

_mesh = plsc.VectorSubcoreMesh(core_axis_name="c", subcore_axis_name="s")


@functools.partial(
    pl.kernel,
    mesh=_mesh,
    out_type=jax.ShapeDtypeStruct((_NC, _NROWS, _D), jnp.float32),
    scratch_types=[
        pltpu.VMEM((_CH, _CHUNK), jnp.int32),
        pltpu.VMEM((_CHUNK, _D), jnp.float32),
        pltpu.VMEM((16, _D), jnp.float32),
        pltpu.VMEM_SHARED((_NROWS, _D), jnp.float32),
    ],
)
def _deg_kernel(dst_hbm, out_hbm, idx_v, ones_v, zero_v, acc):
    c = lax.axis_index("c")
    s = lax.axis_index("s")
    wid = c * _NS + s

    def fill(r, carry):
        for k in range(_D // 16):
            ones_v[r, pl.ds(k * 16, 16)] = jnp.ones((16,), jnp.float32)
        return carry

    lax.fori_loop(0, _CHUNK, fill, 0)

    def fz(r, carry):
        for k in range(_D // 16):
            zero_v[r, pl.ds(k * 16, 16)] = jnp.zeros((16,), jnp.float32)
        return carry

    lax.fori_loop(0, 16, fz, 0)

    def zloop(i, carry):
        pltpu.sync_copy(zero_v, acc.at[pl.ds(s * _STRIPE + i * 16, 16)])
        return carry

    lax.fori_loop(0, _STRIPE // 16, zloop, 0)
    pltpu.sync_copy(dst_hbm.at[pl.ds(wid * _CH, _CH)], idx_v)
    plsc.subcore_barrier()

    def eloop(j, carry):
        pltpu.sync_copy(ones_v, acc.at[idx_v.at[j]], add=True)
        return carry

    lax.fori_loop(0, _CH, eloop, 0)
    plsc.subcore_barrier()

    @pl.when(c == 0)
    def _():
        pltpu.sync_copy(acc.at[pl.ds(s * _STRIPE, _STRIPE)],
                        out_hbm.at[0, pl.ds(s * _STRIPE, _STRIPE)])

    @pl.when(c == 1)
    def _():
        pltpu.sync_copy(acc.at[pl.ds(s * _STRIPE, _STRIPE)],
                        out_hbm.at[1, pl.ds(s * _STRIPE, _STRIPE)])


@functools.partial(
    pl.kernel,
    mesh=_mesh,
    out_type=jax.ShapeDtypeStruct((_NC, _NROWS, _D), jnp.float32),
    scratch_types=[
        pltpu.VMEM((_CH, _CHUNK), jnp.int32),
        pltpu.VMEM((_CH, _CHUNK), jnp.int32),
        pltpu.VMEM((_CHUNK, _D), jnp.float32),
        pltpu.VMEM((16, _D), jnp.float32),
        pltpu.VMEM_SHARED((_NROWS, _D), jnp.float32),
        pltpu.SemaphoreType.DMA,
    ],
)
def _prop_kernel(g_hbm, src_hbm, dst_hbm, out_hbm, srcv, dstv, rows_v, zero_v, acc, sem):
    c = lax.axis_index("c")
    s = lax.axis_index("s")
    wid = c * _NS + s

    def fz(r, carry):
        for k in range(_D // 16):
            zero_v[r, pl.ds(k * 16, 16)] = jnp.zeros((16,), jnp.float32)
        return carry

    lax.fori_loop(0, 16, fz, 0)

    def zloop(i, carry):
        pltpu.sync_copy(zero_v, acc.at[pl.ds(s * _STRIPE + i * 16, 16)])
        return carry

    lax.fori_loop(0, _STRIPE // 16, zloop, 0)
    pltpu.sync_copy(src_hbm.at[pl.ds(wid * _CH, _CH)], srcv)
    pltpu.sync_copy(dst_hbm.at[pl.ds(wid * _CH, _CH)], dstv)
    plsc.subcore_barrier()

    def eloop(j, carry):
        pltpu.async_copy(g_hbm.at[srcv.at[j]], rows_v, sem).wait()
        pltpu.sync_copy(rows_v, acc.at[dstv.at[j]], add=True)
        return carry

    lax.fori_loop(0, _CH, eloop, 0)
    plsc.subcore_barrier()

    @pl.when(c == 0)
    def _():
        pltpu.sync_copy(acc.at[pl.ds(s * _STRIPE, _STRIPE)],
                        out_hbm.at[0, pl.ds(s * _STRIPE, _STRIPE)])

    @pl.when(c == 1)
    def _():
        pltpu.sync_copy(acc.at[pl.ds(s * _STRIPE, _STRIPE)],
                        out_hbm.at[1, pl.ds(s * _STRIPE, _STRIPE)])


def _prep_body(x_ref, w1_ref, degp_ref, g_ref, dinv_ref):
    degp = degp_ref[...]
    deg = degp[0, :, 0:1] + degp[1, :, 0:1] + 1.0
    dinv = lax.rsqrt(deg)
    rows = lax.broadcasted_iota(jnp.int32, (_NROWS, 1), 0)
    dinv = jnp.where(rows < _N, dinv, 0.0)
    dinvb = jnp.broadcast_to(dinv, (_NROWS, _D))
    dinv_ref[...] = dinvb
    g_ref[...] = dinvb * jnp.dot(x_ref[...], w1_ref[...],
                                 preferred_element_type=jnp.float32)


_prep = pl.pallas_call(
    _prep_body,
    out_shape=[
        jax.ShapeDtypeStruct((_NROWS, _D), jnp.float32),
        jax.ShapeDtypeStruct((_NROWS, _D), jnp.float32),
    ],
)


def _layer_body(acc_ref, gprev_ref, dinv_ref, b_ref, w_ref, gnext_ref):
    a = acc_ref[0] + acc_ref[1]
    dinvb = dinv_ref[...]
    h = jnp.maximum(b_ref[...] + dinvb * (a + gprev_ref[...]), 0.0)
    gnext_ref[...] = dinvb * jnp.dot(h, w_ref[...],
                                     preferred_element_type=jnp.float32)


_layer = pl.pallas_call(
    _layer_body,
    out_shape=jax.ShapeDtypeStruct((_NROWS, _D), jnp.float32),
)


def _final_body(acc_ref, g3_ref, dinv_ref, b3_ref, batch_ref, wl1_ref, bl1_ref,
                wl2_ref, bl2_ref, out_ref):
    a = acc_ref[0] + acc_ref[1]
    h3 = jnp.maximum(b3_ref[...] + dinv_ref[...] * (a + g3_ref[...]), 0.0)
    gid = lax.broadcasted_iota(jnp.int32, (_NROWS, _G), 1)
    S = (batch_ref[...] == gid).astype(jnp.float32)
    ps = lax.dot_general(S, h3, (((0,), (0,)), ((), ())),
                         preferred_element_type=jnp.float32)
    cnt = lax.dot_general(S, jnp.ones((_NROWS, 1), jnp.float32),
                          (((0,), (0,)), ((), ())),
                          preferred_element_type=jnp.float32)
    pooled = ps / jnp.maximum(cnt, 1.0)
    hh = jnp.maximum(jnp.dot(pooled, wl1_ref[...],
                             preferred_element_type=jnp.float32) + bl1_ref[...], 0.0)
    logits = jnp.dot(hh, wl2_ref[...],
                     preferred_element_type=jnp.float32) + bl2_ref[...]
    m = jnp.max(logits, axis=-1, keepdims=True)
    e = jnp.exp(logits - m)
    out_ref[...] = (logits - m) - jnp.log(jnp.sum(e, axis=-1, keepdims=True))


_final = pl.pallas_call(
    _final_body,
    out_shape=jax.ShapeDtypeStruct((_G, _C), jnp.float32),
)


def kernel(x, edge_index, batch, W1, b1, W2, b2, W3, b3, Wl1, bl1, Wl2, bl2):
    src = edge_index[0]
    dst = edge_index[1]
    npad = _EPAD - _E
    padi = (jnp.arange(npad, dtype=jnp.int32) % (_NROWS - _N)) + _N
    src_p = jnp.concatenate([src, padi]).reshape(_EPAD // _CHUNK, _CHUNK)
    dst_p = jnp.concatenate([dst, padi]).reshape(_EPAD // _CHUNK, _CHUNK)
    x_p = jnp.pad(x, ((0, _NROWS - _N), (0, 0)))
    batch_p = jnp.concatenate(
        [batch, jnp.full((_NROWS - _N,), _G, jnp.int32)]).reshape(_NROWS, 1)

    degp = _deg_kernel(dst_p)
    g1, dinvb = _prep(x_p, W1, degp)
    a1 = _prop_kernel(g1, src_p, dst_p)
    g2 = _layer(a1, g1, dinvb, b1.reshape(1, _D), W2)
    a2 = _prop_kernel(g2, src_p, dst_p)
    g3 = _layer(a2, g2, dinvb, b2.reshape(1, _D), W3)
    a3 = _prop_kernel(g3, src_p, dst_p)
    return _final(a3, g3, dinvb, b3.reshape(1, _D), batch_p,
                  Wl1, bl1.reshape(1, _D), Wl2, bl2.reshape(1, _C))


# speedup vs baseline: 25.3327x; 1.3977x over previous
import functools

import jax
import jax.numpy as jnp
from jax import lax
from jax.experimental import pallas as pl
from jax.experimental.pallas import tpu as pltpu
from jax.experimental.pallas import tpu_sc as plsc

_N = 10000
_E = 320000
_D = 128
_G = 128
_C = 16

_NC = 2
_NS = 16
_NW = _NC * _NS
_CHUNK = 128
_CH = 80
_PW = _CH * _CHUNK
_EPAD = _NW * _PW
_NROWS = 10240
_STRIPE = _NROWS // _NS

_mesh = plsc.VectorSubcoreMesh(core_axis_name="c", subcore_axis_name="s")


@functools.partial(
    pl.kernel,
    mesh=_mesh,
    out_type=jax.ShapeDtypeStruct((_NC, _NROWS, _D), jnp.float32),
    scratch_types=[
        pltpu.VMEM((_CH, _CHUNK), jnp.int32),
        pltpu.VMEM((_CHUNK, _D), jnp.float32),
        pltpu.VMEM((16, _D), jnp.float32),
        pltpu.VMEM_SHARED((_NROWS, _D), jnp.float32),
    ],
)
def _deg_kernel(dst_hbm, out_hbm, idx_v, ones_v, zero_v, acc):
    c = lax.axis_index("c")
    s = lax.axis_index("s")
    wid = c * _NS + s

    def fill(r, carry):
        for k in range(_D // 16):
            ones_v[r, pl.ds(k * 16, 16)] = jnp.ones((16,), jnp.float32)
        return carry

    lax.fori_loop(0, _CHUNK, fill, 0)

    def fz(r, carry):
        for k in range(_D // 16):
            zero_v[r, pl.ds(k * 16, 16)] = jnp.zeros((16,), jnp.float32)
        return carry

    lax.fori_loop(0, 16, fz, 0)

    def zloop(i, carry):
        pltpu.sync_copy(zero_v, acc.at[pl.ds(s * _STRIPE + i * 16, 16)])
        return carry

    lax.fori_loop(0, _STRIPE // 16, zloop, 0)
    pltpu.sync_copy(dst_hbm.at[pl.ds(wid * _CH, _CH)], idx_v)
    plsc.subcore_barrier()

    def eloop(j, carry):
        pltpu.sync_copy(ones_v, acc.at[idx_v.at[j]], add=True)
        return carry

    lax.fori_loop(0, _CH, eloop, 0)
    plsc.subcore_barrier()

    @pl.when(c == 0)
    def _():
        pltpu.sync_copy(acc.at[pl.ds(s * _STRIPE, _STRIPE)],
                        out_hbm.at[0, pl.ds(s * _STRIPE, _STRIPE)])

    @pl.when(c == 1)
    def _():
        pltpu.sync_copy(acc.at[pl.ds(s * _STRIPE, _STRIPE)],
                        out_hbm.at[1, pl.ds(s * _STRIPE, _STRIPE)])


@functools.partial(
    pl.kernel,
    mesh=_mesh,
    out_type=jax.ShapeDtypeStruct((_NC, _NROWS, _D), jnp.float32),
    scratch_types=[
        pltpu.VMEM((_CH, _CHUNK), jnp.int32),
        pltpu.VMEM((8, _CHUNK), jnp.int32),
        pltpu.VMEM((_CHUNK, _D), jnp.float32),
        pltpu.VMEM((_CHUNK, _D), jnp.float32),
        pltpu.VMEM((16, _D), jnp.float32),
        pltpu.VMEM_SHARED((_NROWS, _D), jnp.float32),
        pltpu.SemaphoreType.DMA,
        pltpu.SemaphoreType.DMA,
    ],
)
def _prop_kernel(g_hbm, src_hbm, dst_hbm, out_hbm, srcv, dstv, bufa, bufb, zero_v, acc, sema, semb):
    c = lax.axis_index("c")
    s = lax.axis_index("s")
    wid = c * _NS + s

    def fz(r, carry):
        for k in range(_D // 16):
            zero_v[r, pl.ds(k * 16, 16)] = jnp.zeros((16,), jnp.float32)
        return carry

    lax.fori_loop(0, 16, fz, 0)

    def zloop(i, carry):
        pltpu.sync_copy(zero_v, acc.at[pl.ds(s * _STRIPE + i * 16, 16)])
        return carry

    lax.fori_loop(0, _STRIPE // 16, zloop, 0)
    pltpu.sync_copy(src_hbm.at[pl.ds(wid * _CH, _CH)], srcv)
    plsc.subcore_barrier()

    def drain(buf, sem):
        pltpu.make_async_copy(g_hbm.at[pl.ds(0, _CHUNK)], buf, sem).wait()

    pltpu.async_copy(g_hbm.at[srcv.at[0]], bufa, sema)

    def gloop(g8, carry):
        pltpu.sync_copy(dst_hbm.at[pl.ds(wid * _CH + g8 * 8, 8)], dstv)
        for k in range(4):
            j0 = g8 * 8 + 2 * k
            pltpu.async_copy(g_hbm.at[srcv.at[j0 + 1]], bufb, semb)
            drain(bufa, sema)
            pltpu.sync_copy(bufa, acc.at[dstv.at[2 * k]], add=True)
            jn = jnp.minimum(j0 + 2, _CH - 1)
            pltpu.async_copy(g_hbm.at[srcv.at[jn]], bufa, sema)
            drain(bufb, semb)
            pltpu.sync_copy(bufb, acc.at[dstv.at[2 * k + 1]], add=True)
        return carry

    lax.fori_loop(0, _CH // 8, gloop, 0)
    drain(bufa, sema)
    plsc.subcore_barrier()

    @pl.when(c == 0)
    def _():
        pltpu.sync_copy(acc.at[pl.ds(s * _STRIPE, _STRIPE)],
                        out_hbm.at[0, pl.ds(s * _STRIPE, _STRIPE)])

    @pl.when(c == 1)
    def _():
        pltpu.sync_copy(acc.at[pl.ds(s * _STRIPE, _STRIPE)],
                        out_hbm.at[1, pl.ds(s * _STRIPE, _STRIPE)])


def _prep_body(x_ref, w1_ref, degp_ref, g_ref, dinv_ref):
    degp = degp_ref[...]
    deg = degp[0, :, 0:1] + degp[1, :, 0:1] + 1.0
    dinv = lax.rsqrt(deg)
    rows = lax.broadcasted_iota(jnp.int32, (_NROWS, 1), 0)
    dinv = jnp.where(rows < _N, dinv, 0.0)
    dinvb = jnp.broadcast_to(dinv, (_NROWS, _D))
    dinv_ref[...] = dinvb
    g_ref[...] = dinvb * jnp.dot(x_ref[...], w1_ref[...],
                                 preferred_element_type=jnp.float32)


_prep = pl.pallas_call(
    _prep_body,
    out_shape=[
        jax.ShapeDtypeStruct((_NROWS, _D), jnp.float32),
        jax.ShapeDtypeStruct((_NROWS, _D), jnp.float32),
    ],
)


def _layer_body(acc_ref, gprev_ref, dinv_ref, b_ref, w_ref, gnext_ref):
    a = acc_ref[0] + acc_ref[1]
    dinvb = dinv_ref[...]
    h = jnp.maximum(b_ref[...] + dinvb * (a + gprev_ref[...]), 0.0)
    gnext_ref[...] = dinvb * jnp.dot(h, w_ref[...],
                                     preferred_element_type=jnp.float32)


_layer = pl.pallas_call(
    _layer_body,
    out_shape=jax.ShapeDtypeStruct((_NROWS, _D), jnp.float32),
)


def _final_body(acc_ref, g3_ref, dinv_ref, b3_ref, batch_ref, wl1_ref, bl1_ref,
                wl2_ref, bl2_ref, out_ref):
    a = acc_ref[0] + acc_ref[1]
    h3 = jnp.maximum(b3_ref[...] + dinv_ref[...] * (a + g3_ref[...]), 0.0)
    gid = lax.broadcasted_iota(jnp.int32, (_NROWS, _G), 1)
    S = (batch_ref[...] == gid).astype(jnp.float32)
    ps = lax.dot_general(S, h3, (((0,), (0,)), ((), ())),
                         preferred_element_type=jnp.float32)
    cnt = lax.dot_general(S, jnp.ones((_NROWS, 1), jnp.float32),
                          (((0,), (0,)), ((), ())),
                          preferred_element_type=jnp.float32)
    pooled = ps / jnp.maximum(cnt, 1.0)
    hh = jnp.maximum(jnp.dot(pooled, wl1_ref[...],
                             preferred_element_type=jnp.float32) + bl1_ref[...], 0.0)
    logits = jnp.dot(hh, wl2_ref[...],
                     preferred_element_type=jnp.float32) + bl2_ref[...]
    m = jnp.max(logits, axis=-1, keepdims=True)
    e = jnp.exp(logits - m)
    out_ref[...] = (logits - m) - jnp.log(jnp.sum(e, axis=-1, keepdims=True))


_final = pl.pallas_call(
    _final_body,
    out_shape=jax.ShapeDtypeStruct((_G, _C), jnp.float32),
)


def kernel(x, edge_index, batch, W1, b1, W2, b2, W3, b3, Wl1, bl1, Wl2, bl2):
    src = edge_index[0]
    dst = edge_index[1]
    npad = _EPAD - _E
    padi = (jnp.arange(npad, dtype=jnp.int32) % (_NROWS - _N)) + _N
    src_p = jnp.concatenate([src, padi]).reshape(_EPAD // _CHUNK, _CHUNK)
    dst_p = jnp.concatenate([dst, padi]).reshape(_EPAD // _CHUNK, _CHUNK)
    x_p = jnp.pad(x, ((0, _NROWS - _N), (0, 0)))
    batch_p = jnp.concatenate(
        [batch, jnp.full((_NROWS - _N,), _G, jnp.int32)]).reshape(_NROWS, 1)

    degp = _deg_kernel(dst_p)
    g1, dinvb = _prep(x_p, W1, degp)
    a1 = _prop_kernel(g1, src_p, dst_p)
    g2 = _layer(a1, g1, dinvb, b1.reshape(1, _D), W2)
    a2 = _prop_kernel(g2, src_p, dst_p)
    g3 = _layer(a2, g2, dinvb, b2.reshape(1, _D), W3)
    a3 = _prop_kernel(g3, src_p, dst_p)
    return _final(a3, g3, dinvb, b3.reshape(1, _D), batch_p,
                  Wl1, bl1.reshape(1, _D), Wl2, bl2.reshape(1, _C))


# speedup vs baseline: 26.0488x; 1.0283x over previous
import functools

import jax
import jax.numpy as jnp
from jax import lax
from jax.experimental import pallas as pl
from jax.experimental.pallas import tpu as pltpu
from jax.experimental.pallas import tpu_sc as plsc

_N = 10000
_E = 320000
_D = 128
_G = 128
_C = 16

_NC = 2
_NS = 16
_NW = _NC * _NS
_CHUNK = 128
_CH = 80
_PW = _CH * _CHUNK
_EPAD = _NW * _PW
_NROWS = 10240
_STRIPE = _NROWS // _NS

_mesh = plsc.VectorSubcoreMesh(core_axis_name="c", subcore_axis_name="s")


@functools.partial(
    pl.kernel,
    mesh=_mesh,
    out_type=jax.ShapeDtypeStruct((_NC, _NROWS, _D), jnp.float32),
    scratch_types=[
        pltpu.VMEM((_CH, _CHUNK), jnp.int32),
        pltpu.VMEM((_CHUNK, _D), jnp.float32),
        pltpu.VMEM((16, _D), jnp.float32),
        pltpu.VMEM_SHARED((_NROWS, _D), jnp.float32),
    ],
)
def _deg_kernel(dst_hbm, out_hbm, idx_v, ones_v, zero_v, acc):
    c = lax.axis_index("c")
    s = lax.axis_index("s")
    wid = c * _NS + s

    def fill(r, carry):
        for k in range(_D // 16):
            ones_v[r, pl.ds(k * 16, 16)] = jnp.ones((16,), jnp.float32)
        return carry

    lax.fori_loop(0, _CHUNK, fill, 0)

    def fz(r, carry):
        for k in range(_D // 16):
            zero_v[r, pl.ds(k * 16, 16)] = jnp.zeros((16,), jnp.float32)
        return carry

    lax.fori_loop(0, 16, fz, 0)

    def zloop(i, carry):
        pltpu.sync_copy(zero_v, acc.at[pl.ds(s * _STRIPE + i * 16, 16)])
        return carry

    lax.fori_loop(0, _STRIPE // 16, zloop, 0)
    pltpu.sync_copy(dst_hbm.at[pl.ds(wid * _CH, _CH)], idx_v)
    plsc.subcore_barrier()

    def eloop(j, carry):
        pltpu.sync_copy(ones_v, acc.at[idx_v.at[j]], add=True)
        return carry

    lax.fori_loop(0, _CH, eloop, 0)
    plsc.subcore_barrier()

    @pl.when(c == 0)
    def _():
        pltpu.sync_copy(acc.at[pl.ds(s * _STRIPE, _STRIPE)],
                        out_hbm.at[0, pl.ds(s * _STRIPE, _STRIPE)])

    @pl.when(c == 1)
    def _():
        pltpu.sync_copy(acc.at[pl.ds(s * _STRIPE, _STRIPE)],
                        out_hbm.at[1, pl.ds(s * _STRIPE, _STRIPE)])


@functools.partial(
    pl.kernel,
    mesh=_mesh,
    out_type=jax.ShapeDtypeStruct((_NC, _NROWS, _D), jnp.float32),
    scratch_types=[
        pltpu.VMEM((_CH, _CHUNK), jnp.int32),
        pltpu.VMEM((2, 8, _CHUNK), jnp.int32),
        pltpu.VMEM((_CHUNK, _D), jnp.float32),
        pltpu.VMEM((_CHUNK, _D), jnp.float32),
        pltpu.VMEM((16, _D), jnp.float32),
        pltpu.VMEM_SHARED((_NROWS, _D), jnp.float32),
        pltpu.SemaphoreType.DMA,
        pltpu.SemaphoreType.DMA,
        pltpu.SemaphoreType.DMA,
    ],
)
def _prop_kernel(g_hbm, src_hbm, dst_hbm, out_hbm, srcv, dstv, bufa, bufb, zero_v, acc,
                 sema, semb, semd):
    c = lax.axis_index("c")
    s = lax.axis_index("s")
    wid = c * _NS + s

    def fz(r, carry):
        for k in range(_D // 16):
            zero_v[r, pl.ds(k * 16, 16)] = jnp.zeros((16,), jnp.float32)
        return carry

    lax.fori_loop(0, 16, fz, 0)

    def zloop(i, carry):
        pltpu.sync_copy(zero_v, acc.at[pl.ds(s * _STRIPE + i * 16, 16)])
        return carry

    lax.fori_loop(0, _STRIPE // 16, zloop, 0)
    pltpu.sync_copy(src_hbm.at[pl.ds(wid * _CH, _CH)], srcv)
    plsc.subcore_barrier()

    def drain(buf, sem):
        pltpu.make_async_copy(g_hbm.at[pl.ds(0, _CHUNK)], buf, sem).wait()

    pltpu.sync_copy(dst_hbm.at[pl.ds(wid * _CH, 8)], dstv.at[0])
    pltpu.async_copy(g_hbm.at[srcv.at[0]], bufa, sema)
    ngrp = _CH // 8

    def gloop(g8, carry):
        par = lax.rem(g8, 2)
        nxt = lax.rem(g8 + 1, 2)
        gn = jnp.minimum(g8 + 1, ngrp - 1)
        pltpu.async_copy(dst_hbm.at[pl.ds(wid * _CH + gn * 8, 8)], dstv.at[nxt], semd)
        for k in range(4):
            j0 = g8 * 8 + 2 * k
            pltpu.async_copy(g_hbm.at[srcv.at[j0 + 1]], bufb, semb)
            drain(bufa, sema)
            pltpu.sync_copy(bufa, acc.at[dstv.at[par, 2 * k]], add=True)
            jn = jnp.minimum(j0 + 2, _CH - 1)
            pltpu.async_copy(g_hbm.at[srcv.at[jn]], bufa, sema)
            drain(bufb, semb)
            pltpu.sync_copy(bufb, acc.at[dstv.at[par, 2 * k + 1]], add=True)
        pltpu.make_async_copy(dst_hbm.at[pl.ds(0, 8)], dstv.at[nxt], semd).wait()
        return carry

    lax.fori_loop(0, ngrp, gloop, 0)
    drain(bufa, sema)
    plsc.subcore_barrier()

    @pl.when(c == 0)
    def _():
        pltpu.sync_copy(acc.at[pl.ds(s * _STRIPE, _STRIPE)],
                        out_hbm.at[0, pl.ds(s * _STRIPE, _STRIPE)])

    @pl.when(c == 1)
    def _():
        pltpu.sync_copy(acc.at[pl.ds(s * _STRIPE, _STRIPE)],
                        out_hbm.at[1, pl.ds(s * _STRIPE, _STRIPE)])


def _prep_body(x_ref, w1_ref, degp_ref, g_ref, dinv_ref):
    degp = degp_ref[...]
    deg = degp[0, :, 0:1] + degp[1, :, 0:1] + 1.0
    dinv = lax.rsqrt(deg)
    rows = lax.broadcasted_iota(jnp.int32, (_NROWS, 1), 0)
    dinv = jnp.where(rows < _N, dinv, 0.0)
    dinvb = jnp.broadcast_to(dinv, (_NROWS, _D))
    dinv_ref[...] = dinvb
    g_ref[...] = dinvb * jnp.dot(x_ref[...], w1_ref[...],
                                 preferred_element_type=jnp.float32)


_prep = pl.pallas_call(
    _prep_body,
    out_shape=[
        jax.ShapeDtypeStruct((_NROWS, _D), jnp.float32),
        jax.ShapeDtypeStruct((_NROWS, _D), jnp.float32),
    ],
)


def _layer_body(acc_ref, gprev_ref, dinv_ref, b_ref, w_ref, gnext_ref):
    a = acc_ref[0] + acc_ref[1]
    dinvb = dinv_ref[...]
    h = jnp.maximum(b_ref[...] + dinvb * (a + gprev_ref[...]), 0.0)
    gnext_ref[...] = dinvb * jnp.dot(h, w_ref[...],
                                     preferred_element_type=jnp.float32)


_layer = pl.pallas_call(
    _layer_body,
    out_shape=jax.ShapeDtypeStruct((_NROWS, _D), jnp.float32),
)


def _final_body(acc_ref, g3_ref, dinv_ref, b3_ref, batch_ref, wl1_ref, bl1_ref,
                wl2_ref, bl2_ref, out_ref):
    a = acc_ref[0] + acc_ref[1]
    h3 = jnp.maximum(b3_ref[...] + dinv_ref[...] * (a + g3_ref[...]), 0.0)
    gid = lax.broadcasted_iota(jnp.int32, (_NROWS, _G), 1)
    S = (batch_ref[...] == gid).astype(jnp.float32)
    ps = lax.dot_general(S, h3, (((0,), (0,)), ((), ())),
                         preferred_element_type=jnp.float32)
    cnt = lax.dot_general(S, jnp.ones((_NROWS, 1), jnp.float32),
                          (((0,), (0,)), ((), ())),
                          preferred_element_type=jnp.float32)
    pooled = ps / jnp.maximum(cnt, 1.0)
    hh = jnp.maximum(jnp.dot(pooled, wl1_ref[...],
                             preferred_element_type=jnp.float32) + bl1_ref[...], 0.0)
    logits = jnp.dot(hh, wl2_ref[...],
                     preferred_element_type=jnp.float32) + bl2_ref[...]
    m = jnp.max(logits, axis=-1, keepdims=True)
    e = jnp.exp(logits - m)
    out_ref[...] = (logits - m) - jnp.log(jnp.sum(e, axis=-1, keepdims=True))


_final = pl.pallas_call(
    _final_body,
    out_shape=jax.ShapeDtypeStruct((_G, _C), jnp.float32),
)


def kernel(x, edge_index, batch, W1, b1, W2, b2, W3, b3, Wl1, bl1, Wl2, bl2):
    src = edge_index[0]
    dst = edge_index[1]
    npad = _EPAD - _E
    padi = (jnp.arange(npad, dtype=jnp.int32) % (_NROWS - _N)) + _N
    src_p = jnp.concatenate([src, padi]).reshape(_EPAD // _CHUNK, _CHUNK)
    dst_p = jnp.concatenate([dst, padi]).reshape(_EPAD // _CHUNK, _CHUNK)
    x_p = jnp.pad(x, ((0, _NROWS - _N), (0, 0)))
    batch_p = jnp.concatenate(
        [batch, jnp.full((_NROWS - _N,), _G, jnp.int32)]).reshape(_NROWS, 1)

    degp = _deg_kernel(dst_p)
    g1, dinvb = _prep(x_p, W1, degp)
    a1 = _prop_kernel(g1, src_p, dst_p)
    g2 = _layer(a1, g1, dinvb, b1.reshape(1, _D), W2)
    a2 = _prop_kernel(g2, src_p, dst_p)
    g3 = _layer(a2, g2, dinvb, b2.reshape(1, _D), W3)
    a3 = _prop_kernel(g3, src_p, dst_p)
    return _final(a3, g3, dinvb, b3.reshape(1, _D), batch_p,
                  Wl1, bl1.reshape(1, _D), Wl2, bl2.reshape(1, _C))


# speedup vs baseline: 26.5070x; 1.0176x over previous
import functools

import jax
import jax.numpy as jnp
from jax import lax
from jax.experimental import pallas as pl
from jax.experimental.pallas import tpu as pltpu
from jax.experimental.pallas import tpu_sc as plsc

_N = 10000
_E = 320000
_D = 128
_G = 128
_C = 16

_NC = 2
_NS = 16
_NW = _NC * _NS
_CHUNK = 128
_CH = 80
_PW = _CH * _CHUNK
_EPAD = _NW * _PW
_NROWS = 10240
_STRIPE = _NROWS // _NS

_mesh = plsc.VectorSubcoreMesh(core_axis_name="c", subcore_axis_name="s")


@functools.partial(
    pl.kernel,
    mesh=_mesh,
    out_type=jax.ShapeDtypeStruct((_NC, _NROWS, _D), jnp.float32),
    scratch_types=[
        pltpu.VMEM((_CH, _CHUNK), jnp.int32),
        pltpu.VMEM((_CHUNK, _D), jnp.float32),
        pltpu.VMEM((16, _D), jnp.float32),
        pltpu.VMEM_SHARED((_NROWS, _D), jnp.float32),
        pltpu.SemaphoreType.DMA,
    ],
)
def _deg_kernel(dst_hbm, out_hbm, idx_v, ones_v, zero_v, acc, sem):
    c = lax.axis_index("c")
    s = lax.axis_index("s")
    wid = c * _NS + s

    def fill(r, carry):
        for k in range(_D // 16):
            ones_v[r, pl.ds(k * 16, 16)] = jnp.ones((16,), jnp.float32)
        return carry

    lax.fori_loop(0, _CHUNK, fill, 0)

    def fz(r, carry):
        for k in range(_D // 16):
            zero_v[r, pl.ds(k * 16, 16)] = jnp.zeros((16,), jnp.float32)
        return carry

    lax.fori_loop(0, 16, fz, 0)

    def zloop(i, carry):
        for k in range(8):
            pltpu.async_copy(zero_v, acc.at[pl.ds(s * _STRIPE + (i * 8 + k) * 16, 16)], sem)
        for k in range(8):
            pltpu.make_async_copy(zero_v, acc.at[pl.ds(s * _STRIPE, 16)], sem).wait()
        return carry

    lax.fori_loop(0, _STRIPE // 16 // 8, zloop, 0)
    pltpu.sync_copy(dst_hbm.at[pl.ds(wid * _CH, _CH)], idx_v)
    plsc.subcore_barrier()

    def eloop(gi, carry):
        for k in range(8):
            pltpu.async_copy(ones_v, acc.at[idx_v.at[gi * 8 + k]], sem, add=True)
        for k in range(8):
            pltpu.make_async_copy(ones_v, acc.at[pl.ds(s * _STRIPE, _CHUNK)], sem).wait()
        return carry

    lax.fori_loop(0, _CH // 8, eloop, 0)
    plsc.subcore_barrier()

    @pl.when(c == 0)
    def _():
        pltpu.sync_copy(acc.at[pl.ds(s * _STRIPE, _STRIPE)],
                        out_hbm.at[0, pl.ds(s * _STRIPE, _STRIPE)])

    @pl.when(c == 1)
    def _():
        pltpu.sync_copy(acc.at[pl.ds(s * _STRIPE, _STRIPE)],
                        out_hbm.at[1, pl.ds(s * _STRIPE, _STRIPE)])


@functools.partial(
    pl.kernel,
    mesh=_mesh,
    out_type=jax.ShapeDtypeStruct((_NC, _NROWS, _D), jnp.float32),
    scratch_types=[
        pltpu.VMEM((_CH, _CHUNK), jnp.int32),
        pltpu.VMEM((2, 8, _CHUNK), jnp.int32),
        pltpu.VMEM((_CHUNK, _D), jnp.float32),
        pltpu.VMEM((_CHUNK, _D), jnp.float32),
        pltpu.VMEM((16, _D), jnp.float32),
        pltpu.VMEM_SHARED((_NROWS, _D), jnp.float32),
        pltpu.SemaphoreType.DMA,
        pltpu.SemaphoreType.DMA,
        pltpu.SemaphoreType.DMA,
    ],
)
def _prop_kernel(g_hbm, src_hbm, dst_hbm, out_hbm, srcv, dstv, bufa, bufb, zero_v, acc,
                 sema, semb, semd):
    c = lax.axis_index("c")
    s = lax.axis_index("s")
    wid = c * _NS + s

    def fz(r, carry):
        for k in range(_D // 16):
            zero_v[r, pl.ds(k * 16, 16)] = jnp.zeros((16,), jnp.float32)
        return carry

    lax.fori_loop(0, 16, fz, 0)

    def zloop(i, carry):
        for k in range(8):
            pltpu.async_copy(zero_v, acc.at[pl.ds(s * _STRIPE + (i * 8 + k) * 16, 16)], sema)
        for k in range(8):
            pltpu.make_async_copy(zero_v, acc.at[pl.ds(s * _STRIPE, 16)], sema).wait()
        return carry

    lax.fori_loop(0, _STRIPE // 16 // 8, zloop, 0)
    pltpu.sync_copy(src_hbm.at[pl.ds(wid * _CH, _CH)], srcv)
    plsc.subcore_barrier()

    def drain(buf, sem):
        pltpu.make_async_copy(g_hbm.at[pl.ds(0, _CHUNK)], buf, sem).wait()

    pltpu.sync_copy(dst_hbm.at[pl.ds(wid * _CH, 8)], dstv.at[0])
    pltpu.async_copy(g_hbm.at[srcv.at[0]], bufa, sema)
    ngrp = _CH // 8

    def gloop(g8, carry):
        par = lax.rem(g8, 2)
        nxt = lax.rem(g8 + 1, 2)
        gn = jnp.minimum(g8 + 1, ngrp - 1)
        pltpu.async_copy(dst_hbm.at[pl.ds(wid * _CH + gn * 8, 8)], dstv.at[nxt], semd)
        for k in range(4):
            j0 = g8 * 8 + 2 * k
            pltpu.async_copy(g_hbm.at[srcv.at[j0 + 1]], bufb, semb)
            drain(bufa, sema)
            pltpu.sync_copy(bufa, acc.at[dstv.at[par, 2 * k]], add=True)
            jn = jnp.minimum(j0 + 2, _CH - 1)
            pltpu.async_copy(g_hbm.at[srcv.at[jn]], bufa, sema)
            drain(bufb, semb)
            pltpu.sync_copy(bufb, acc.at[dstv.at[par, 2 * k + 1]], add=True)
        pltpu.make_async_copy(dst_hbm.at[pl.ds(0, 8)], dstv.at[nxt], semd).wait()
        return carry

    lax.fori_loop(0, ngrp, gloop, 0)
    drain(bufa, sema)
    plsc.subcore_barrier()

    @pl.when(c == 0)
    def _():
        pltpu.sync_copy(acc.at[pl.ds(s * _STRIPE, _STRIPE)],
                        out_hbm.at[0, pl.ds(s * _STRIPE, _STRIPE)])

    @pl.when(c == 1)
    def _():
        pltpu.sync_copy(acc.at[pl.ds(s * _STRIPE, _STRIPE)],
                        out_hbm.at[1, pl.ds(s * _STRIPE, _STRIPE)])


def _prep_body(x_ref, w1_ref, degp_ref, g_ref, dinv_ref):
    degp = degp_ref[...]
    deg = degp[0, :, 0:1] + degp[1, :, 0:1] + 1.0
    dinv = lax.rsqrt(deg)
    rows = lax.broadcasted_iota(jnp.int32, (_NROWS, 1), 0)
    dinv = jnp.where(rows < _N, dinv, 0.0)
    dinvb = jnp.broadcast_to(dinv, (_NROWS, _D))
    dinv_ref[...] = dinvb
    g_ref[...] = dinvb * jnp.dot(x_ref[...], w1_ref[...],
                                 preferred_element_type=jnp.float32)


_prep = pl.pallas_call(
    _prep_body,
    out_shape=[
        jax.ShapeDtypeStruct((_NROWS, _D), jnp.float32),
        jax.ShapeDtypeStruct((_NROWS, _D), jnp.float32),
    ],
)


def _layer_body(acc_ref, gprev_ref, dinv_ref, b_ref, w_ref, gnext_ref):
    a = acc_ref[0] + acc_ref[1]
    dinvb = dinv_ref[...]
    h = jnp.maximum(b_ref[...] + dinvb * (a + gprev_ref[...]), 0.0)
    gnext_ref[...] = dinvb * jnp.dot(h, w_ref[...],
                                     preferred_element_type=jnp.float32)


_layer = pl.pallas_call(
    _layer_body,
    out_shape=jax.ShapeDtypeStruct((_NROWS, _D), jnp.float32),
)


def _final_body(acc_ref, g3_ref, dinv_ref, b3_ref, batch_ref, wl1_ref, bl1_ref,
                wl2_ref, bl2_ref, out_ref):
    a = acc_ref[0] + acc_ref[1]
    h3 = jnp.maximum(b3_ref[...] + dinv_ref[...] * (a + g3_ref[...]), 0.0)
    gid = lax.broadcasted_iota(jnp.int32, (_NROWS, _G), 1)
    S = (batch_ref[...] == gid).astype(jnp.float32)
    ps = lax.dot_general(S, h3, (((0,), (0,)), ((), ())),
                         preferred_element_type=jnp.float32)
    cnt = lax.dot_general(S, jnp.ones((_NROWS, 1), jnp.float32),
                          (((0,), (0,)), ((), ())),
                          preferred_element_type=jnp.float32)
    pooled = ps / jnp.maximum(cnt, 1.0)
    hh = jnp.maximum(jnp.dot(pooled, wl1_ref[...],
                             preferred_element_type=jnp.float32) + bl1_ref[...], 0.0)
    logits = jnp.dot(hh, wl2_ref[...],
                     preferred_element_type=jnp.float32) + bl2_ref[...]
    m = jnp.max(logits, axis=-1, keepdims=True)
    e = jnp.exp(logits - m)
    out_ref[...] = (logits - m) - jnp.log(jnp.sum(e, axis=-1, keepdims=True))


_final = pl.pallas_call(
    _final_body,
    out_shape=jax.ShapeDtypeStruct((_G, _C), jnp.float32),
)


def kernel(x, edge_index, batch, W1, b1, W2, b2, W3, b3, Wl1, bl1, Wl2, bl2):
    src = edge_index[0]
    dst = edge_index[1]
    npad = _EPAD - _E
    padi = (jnp.arange(npad, dtype=jnp.int32) % (_NROWS - _N)) + _N
    src_p = jnp.concatenate([src, padi]).reshape(_EPAD // _CHUNK, _CHUNK)
    dst_p = jnp.concatenate([dst, padi]).reshape(_EPAD // _CHUNK, _CHUNK)
    x_p = jnp.pad(x, ((0, _NROWS - _N), (0, 0)))
    batch_p = jnp.concatenate(
        [batch, jnp.full((_NROWS - _N,), _G, jnp.int32)]).reshape(_NROWS, 1)

    degp = _deg_kernel(dst_p)
    g1, dinvb = _prep(x_p, W1, degp)
    a1 = _prop_kernel(g1, src_p, dst_p)
    g2 = _layer(a1, g1, dinvb, b1.reshape(1, _D), W2)
    a2 = _prop_kernel(g2, src_p, dst_p)
    g3 = _layer(a2, g2, dinvb, b2.reshape(1, _D), W3)
    a3 = _prop_kernel(g3, src_p, dst_p)
    return _final(a3, g3, dinvb, b3.reshape(1, _D), batch_p,
                  Wl1, bl1.reshape(1, _D), Wl2, bl2.reshape(1, _C))


# speedup vs baseline: 29.5842x; 1.1161x over previous
import functools

import jax
import jax.numpy as jnp
from jax import lax
from jax.experimental import pallas as pl
from jax.experimental.pallas import tpu as pltpu
from jax.experimental.pallas import tpu_sc as plsc

_N = 10000
_E = 320000
_D = 128
_G = 128
_C = 16

_NC = 2
_NS = 16
_NW = _NC * _NS
_CHUNK = 128
_CH = 80
_PW = _CH * _CHUNK
_EPAD = _NW * _PW
_NROWS = 10240
_STRIPE = _NROWS // _NS

_mesh = plsc.VectorSubcoreMesh(core_axis_name="c", subcore_axis_name="s")


@functools.partial(
    pl.kernel,
    mesh=_mesh,
    compiler_params=pltpu.CompilerParams(needs_layout_passes=False),
    out_type=jax.ShapeDtypeStruct((_NC, _NROWS, _D), jnp.float32),
    scratch_types=[
        pltpu.VMEM((_CH, _CHUNK), jnp.int32),
        pltpu.VMEM((_NROWS,), jnp.float32),
        pltpu.VMEM((_NS, _STRIPE), jnp.float32),
        pltpu.VMEM((_STRIPE,), jnp.float32),
        pltpu.VMEM((_CHUNK, _D), jnp.float32),
        pltpu.VMEM_SHARED((_NS, _NROWS), jnp.float32),
    ],
)
def _deg_kernel(dst_hbm, out_hbm, idx_v, hist, gbuf, dsum, brow, shared):
    c = lax.axis_index("c")
    s = lax.axis_index("s")
    wid = c * _NS + s

    def hz(r, carry):
        hist[pl.ds(r * 16, 16)] = jnp.zeros((16,), jnp.float32)
        return carry

    lax.fori_loop(0, _NROWS // 16, hz, 0)
    pltpu.sync_copy(dst_hbm.at[pl.ds(wid * _CH, _CH)], idx_v)
    ones16 = jnp.ones((16,), jnp.float32)

    def hloop(j, carry):
        for k in range(_CHUNK // 16):
            idx16 = idx_v[j, pl.ds(k * 16, 16)]
            plsc.addupdate_scatter(hist, [idx16], ones16)
        return carry

    lax.fori_loop(0, _CH, hloop, 0)
    pltpu.sync_copy(hist, shared.at[s])
    plsc.subcore_barrier()
    pltpu.sync_copy(shared.at[:, pl.ds(s * _STRIPE, _STRIPE)], gbuf)

    def rloop(g, carry):
        v = gbuf[0, pl.ds(g * 16, 16)]
        for r in range(1, _NS):
            v = v + gbuf[r, pl.ds(g * 16, 16)]
        dsum[pl.ds(g * 16, 16)] = v
        return carry

    lax.fori_loop(0, _STRIPE // 16, rloop, 0)

    for b in range(_STRIPE // _CHUNK):
        def bloop(r, carry):
            idx16 = jnp.zeros((16,), jnp.int32) + (b * _CHUNK + r)
            row = plsc.load_gather(dsum, [idx16])
            for k in range(_D // 16):
                brow[r, pl.ds(k * 16, 16)] = row
            return carry

        lax.fori_loop(0, _CHUNK, bloop, 0)

        @pl.when(c == 0)
        def _():
            pltpu.sync_copy(brow, out_hbm.at[0, pl.ds(s * _STRIPE + b * _CHUNK, _CHUNK)])

        @pl.when(c == 1)
        def _():
            pltpu.sync_copy(brow, out_hbm.at[1, pl.ds(s * _STRIPE + b * _CHUNK, _CHUNK)])


@functools.partial(
    pl.kernel,
    mesh=_mesh,
    out_type=jax.ShapeDtypeStruct((_NC, _NROWS, _D), jnp.float32),
    scratch_types=[
        pltpu.VMEM((_CH, _CHUNK), jnp.int32),
        pltpu.VMEM((2, 8, _CHUNK), jnp.int32),
        pltpu.VMEM((_CHUNK, _D), jnp.float32),
        pltpu.VMEM((_CHUNK, _D), jnp.float32),
        pltpu.VMEM((16, _D), jnp.float32),
        pltpu.VMEM_SHARED((_NROWS, _D), jnp.float32),
        pltpu.SemaphoreType.DMA,
        pltpu.SemaphoreType.DMA,
        pltpu.SemaphoreType.DMA,
    ],
)
def _prop_kernel(g_hbm, src_hbm, dst_hbm, out_hbm, srcv, dstv, bufa, bufb, zero_v, acc,
                 sema, semb, semd):
    c = lax.axis_index("c")
    s = lax.axis_index("s")
    wid = c * _NS + s

    def fz(r, carry):
        for k in range(_D // 16):
            zero_v[r, pl.ds(k * 16, 16)] = jnp.zeros((16,), jnp.float32)
        return carry

    lax.fori_loop(0, 16, fz, 0)

    def zloop(i, carry):
        for k in range(8):
            pltpu.async_copy(zero_v, acc.at[pl.ds(s * _STRIPE + (i * 8 + k) * 16, 16)], sema)
        for k in range(8):
            pltpu.make_async_copy(zero_v, acc.at[pl.ds(s * _STRIPE, 16)], sema).wait()
        return carry

    lax.fori_loop(0, _STRIPE // 16 // 8, zloop, 0)
    pltpu.sync_copy(src_hbm.at[pl.ds(wid * _CH, _CH)], srcv)

    def drain(buf, sem):
        pltpu.make_async_copy(g_hbm.at[pl.ds(0, _CHUNK)], buf, sem).wait()

    pltpu.sync_copy(dst_hbm.at[pl.ds(wid * _CH, 8)], dstv.at[0])
    pltpu.async_copy(g_hbm.at[srcv.at[0]], bufa, sema)
    plsc.subcore_barrier()
    ngrp = _CH // 8

    def gloop(g8, carry):
        par = lax.rem(g8, 2)
        nxt = lax.rem(g8 + 1, 2)
        gn = jnp.minimum(g8 + 1, ngrp - 1)
        pltpu.async_copy(dst_hbm.at[pl.ds(wid * _CH + gn * 8, 8)], dstv.at[nxt], semd)
        for k in range(4):
            j0 = g8 * 8 + 2 * k
            pltpu.async_copy(g_hbm.at[srcv.at[j0 + 1]], bufb, semb)
            drain(bufa, sema)
            pltpu.sync_copy(bufa, acc.at[dstv.at[par, 2 * k]], add=True)
            jn = jnp.minimum(j0 + 2, _CH - 1)
            pltpu.async_copy(g_hbm.at[srcv.at[jn]], bufa, sema)
            drain(bufb, semb)
            pltpu.sync_copy(bufb, acc.at[dstv.at[par, 2 * k + 1]], add=True)
        pltpu.make_async_copy(dst_hbm.at[pl.ds(0, 8)], dstv.at[nxt], semd).wait()
        return carry

    lax.fori_loop(0, ngrp, gloop, 0)
    drain(bufa, sema)
    plsc.subcore_barrier()

    @pl.when(c == 0)
    def _():
        pltpu.sync_copy(acc.at[pl.ds(s * _STRIPE, _STRIPE)],
                        out_hbm.at[0, pl.ds(s * _STRIPE, _STRIPE)])

    @pl.when(c == 1)
    def _():
        pltpu.sync_copy(acc.at[pl.ds(s * _STRIPE, _STRIPE)],
                        out_hbm.at[1, pl.ds(s * _STRIPE, _STRIPE)])


def _prep_body(x_ref, w1_ref, degp_ref, g_ref, dinv_ref):
    degp = degp_ref[...]
    deg = degp[0, :, 0:1] + degp[1, :, 0:1] + 1.0
    dinv = lax.rsqrt(deg)
    rows = lax.broadcasted_iota(jnp.int32, (_NROWS, 1), 0)
    dinv = jnp.where(rows < _N, dinv, 0.0)
    dinvb = jnp.broadcast_to(dinv, (_NROWS, _D))
    dinv_ref[...] = dinvb
    g_ref[...] = dinvb * jnp.dot(x_ref[...], w1_ref[...],
                                 preferred_element_type=jnp.float32)


_prep = pl.pallas_call(
    _prep_body,
    out_shape=[
        jax.ShapeDtypeStruct((_NROWS, _D), jnp.float32),
        jax.ShapeDtypeStruct((_NROWS, _D), jnp.float32),
    ],
)


def _layer_body(acc_ref, gprev_ref, dinv_ref, b_ref, w_ref, gnext_ref):
    a = acc_ref[0] + acc_ref[1]
    dinvb = dinv_ref[...]
    h = jnp.maximum(b_ref[...] + dinvb * (a + gprev_ref[...]), 0.0)
    gnext_ref[...] = dinvb * jnp.dot(h, w_ref[...],
                                     preferred_element_type=jnp.float32)


_layer = pl.pallas_call(
    _layer_body,
    out_shape=jax.ShapeDtypeStruct((_NROWS, _D), jnp.float32),
)


def _final_body(acc_ref, g3_ref, dinv_ref, b3_ref, batch_ref, wl1_ref, bl1_ref,
                wl2_ref, bl2_ref, out_ref):
    a = acc_ref[0] + acc_ref[1]
    h3 = jnp.maximum(b3_ref[...] + dinv_ref[...] * (a + g3_ref[...]), 0.0)
    gid = lax.broadcasted_iota(jnp.int32, (_NROWS, _G), 1)
    S = (batch_ref[...] == gid).astype(jnp.float32)
    ps = lax.dot_general(S, h3, (((0,), (0,)), ((), ())),
                         preferred_element_type=jnp.float32)
    cnt = lax.dot_general(S, jnp.ones((_NROWS, 1), jnp.float32),
                          (((0,), (0,)), ((), ())),
                          preferred_element_type=jnp.float32)
    pooled = ps / jnp.maximum(cnt, 1.0)
    hh = jnp.maximum(jnp.dot(pooled, wl1_ref[...],
                             preferred_element_type=jnp.float32) + bl1_ref[...], 0.0)
    logits = jnp.dot(hh, wl2_ref[...],
                     preferred_element_type=jnp.float32) + bl2_ref[...]
    m = jnp.max(logits, axis=-1, keepdims=True)
    e = jnp.exp(logits - m)
    out_ref[...] = (logits - m) - jnp.log(jnp.sum(e, axis=-1, keepdims=True))


_final = pl.pallas_call(
    _final_body,
    out_shape=jax.ShapeDtypeStruct((_G, _C), jnp.float32),
)


def kernel(x, edge_index, batch, W1, b1, W2, b2, W3, b3, Wl1, bl1, Wl2, bl2):
    src = edge_index[0]
    dst = edge_index[1]
    npad = _EPAD - _E
    padi = (jnp.arange(npad, dtype=jnp.int32) % (_NROWS - _N)) + _N
    src_p = jnp.concatenate([src, padi]).reshape(_EPAD // _CHUNK, _CHUNK)
    dst_p = jnp.concatenate([dst, padi]).reshape(_EPAD // _CHUNK, _CHUNK)
    x_p = jnp.pad(x, ((0, _NROWS - _N), (0, 0)))
    batch_p = jnp.concatenate(
        [batch, jnp.full((_NROWS - _N,), _G, jnp.int32)]).reshape(_NROWS, 1)

    degp = _deg_kernel(dst_p)
    g1, dinvb = _prep(x_p, W1, degp)
    a1 = _prop_kernel(g1, src_p, dst_p)
    g2 = _layer(a1, g1, dinvb, b1.reshape(1, _D), W2)
    a2 = _prop_kernel(g2, src_p, dst_p)
    g3 = _layer(a2, g2, dinvb, b2.reshape(1, _D), W3)
    a3 = _prop_kernel(g3, src_p, dst_p)
    return _final(a3, g3, dinvb, b3.reshape(1, _D), batch_p,
                  Wl1, bl1.reshape(1, _D), Wl2, bl2.reshape(1, _C))
